# baseline (speedup 1.0000x reference)
import functools

import jax
import jax.numpy as jnp
from jax import lax
from jax.experimental import pallas as pl
from jax.experimental.pallas import tpu as pltpu
from jax.experimental.pallas import tpu_sc as plsc

N = 10000
E = 320000
D_IN = 128
HEADS = 8
PH = 16
HID = 128
NCLS = 40
SLOPE = 0.2

NC = 2
NS = 16
NW = NC * NS
LANES = 16

NP = 10240
BLK = 256
EW = E // NW
B = 40
NCHUNK = EW // B
RPT = NP // NS

_BIG = 1e30


def _leaky(v):
  return jnp.where(v >= 0, v, v * SLOPE)


def _prep_common(h_in, W_ref, b_ref, LR_ref, HA_ref, T_ref, A_ref, d_out):
  h = jnp.dot(h_in, W_ref[...], preferred_element_type=jnp.float32)
  h = h + b_ref[...]
  t = jnp.dot(h, LR_ref[...], preferred_element_type=jnp.float32)
  T_ref[...] = t
  HA_ref[:, 0:d_out] = h
  HA_ref[:, d_out:d_out + 8] = t[:, 0:8]
  HA_ref[:, d_out + 8:d_out + 16] = jnp.zeros((h.shape[0], 8), jnp.float32)
  blockmax = jnp.max(t[:, 0:8], axis=0, keepdims=True)
  cur = jnp.concatenate(
      [blockmax, jnp.full((1, 8), _BIG, jnp.float32)], axis=1)
  i = pl.program_id(0)

  @pl.when(i == 0)
  def _():
    A_ref[...] = cur

  @pl.when(i > 0)
  def _():
    A_ref[...] = jnp.maximum(A_ref[...], cur)


def _tc_prep(p, bb_row, W, b_row, LR, d_in, d_out, first):
  grid = NP // BLK

  def body(p_ref, bb_ref, W_ref, b_ref, LR_ref, HA_ref, T_ref, A_ref):
    if first:
      h_in = p_ref[...]
    else:
      s = p_ref[0] + p_ref[1] + bb_ref[...]
      h_in = jnp.where(s > 0, s, jnp.exp(s) - 1.0)
    _prep_common(h_in, W_ref, b_ref, LR_ref, HA_ref, T_ref, A_ref, d_out)

  in_spec_p = (
      pl.BlockSpec((BLK, d_in), lambda i: (i, 0)) if first
      else pl.BlockSpec((2, BLK, d_in), lambda i: (0, i, 0)))
  return pl.pallas_call(
      body,
      grid=(grid,),
      in_specs=[
          in_spec_p,
          pl.BlockSpec((1, d_in), lambda i: (0, 0)),
          pl.BlockSpec((d_in, d_out), lambda i: (0, 0)),
          pl.BlockSpec((1, d_out), lambda i: (0, 0)),
          pl.BlockSpec((d_out, 16), lambda i: (0, 0)),
      ],
      out_specs=[
          pl.BlockSpec((BLK, d_out + 16), lambda i: (i, 0)),
          pl.BlockSpec((BLK, 16), lambda i: (i, 0)),
          pl.BlockSpec((1, 16), lambda i: (0, 0)),
      ],
      out_shape=[
          jax.ShapeDtypeStruct((NP, d_out + 16), jnp.float32),
          jax.ShapeDtypeStruct((NP, 16), jnp.float32),
          jax.ShapeDtypeStruct((1, 16), jnp.float32),
      ],
  )(p, bb_row, W, b_row, LR)


def _final_body(p_ref, bb_ref, o_ref):
  o_ref[...] = p_ref[0] + p_ref[1] + bb_ref[...]


def _tc_final(p, bb_row, d_out):
  grid = NP // BLK
  return pl.pallas_call(
      _final_body,
      grid=(grid,),
      in_specs=[
          pl.BlockSpec((2, BLK, d_out), lambda i: (0, i, 0)),
          pl.BlockSpec((1, d_out), lambda i: (0, 0)),
      ],
      out_specs=pl.BlockSpec((BLK, d_out), lambda i: (i, 0)),
      out_shape=jax.ShapeDtypeStruct((NP, d_out), jnp.float32),
  )(p, bb_row)


_MESH = plsc.VectorSubcoreMesh(core_axis_name="c", subcore_axis_name="s")


def _edge_w(ts, td, av):
  rot_idx = (lax.iota(jnp.int32, LANES) & 7) + 8
  rot = jnp.take_along_axis(td, rot_idx, axis=0)
  e = _leaky(ts + rot)
  m = _leaky(rot + av)
  return jnp.exp(e - m)


def _passA_body(esrc, edst, T, avec, s_out, src_all, dst_all, tsrc, tdst,
                wbuf, a_v, s_sh, semA, semB, semSA, semSB):
  cid = lax.axis_index("c")
  sid = lax.axis_index("s")
  wid = cid * NS + sid
  tsrcs = [tsrc.at[0], tsrc.at[1]]
  tdsts = [tdst.at[0], tdst.at[1]]
  wbufs = [wbuf.at[0], wbuf.at[1]]
  sems = [semA, semB]
  ssems = [semSA, semSB]

  def zero_row(i, c):
    wbuf[0, i, :] = jnp.zeros((LANES,), jnp.float32)
    return c

  lax.fori_loop(0, B, zero_row, 0)

  def zero_sh(k, c):
    pltpu.sync_copy(wbufs[0], s_sh.at[pl.ds(sid * RPT + k * B, B)])
    return c

  lax.fori_loop(0, RPT // B, zero_sh, 0)
  plsc.subcore_barrier()

  pltpu.sync_copy(avec, a_v)
  av = a_v[:]
  pltpu.sync_copy(esrc.at[wid], src_all)
  pltpu.sync_copy(edst.at[wid], dst_all)

  def fire(ci, b):
    pltpu.async_copy(T.at[src_all.at[ci]], tsrcs[b], sems[b])
    pltpu.async_copy(T.at[dst_all.at[ci]], tdsts[b], sems[b])

  def wait_gather(ci, b):
    pltpu.make_async_copy(T.at[src_all.at[ci]], tsrcs[b], sems[b]).wait()
    pltpu.make_async_copy(T.at[dst_all.at[ci]], tdsts[b], sems[b]).wait()

  def compute(ci, b):
    @plsc.parallel_loop(0, B, step=1, unroll=4)
    def per_edge(i):
      wbuf[b, i, :] = _edge_w(tsrc[b, i, :], tdst[b, i, :], av)

  def fire_sc(ci, b):
    pltpu.async_copy(wbufs[b], s_sh.at[dst_all.at[ci]], ssems[b], add=True)

  def wait_sc(ci, b):
    pltpu.make_async_copy(wbufs[b], s_sh.at[dst_all.at[ci]], ssems[b]).wait()

  fire(0, 0)
  fire(1, 1)
  wait_gather(0, 0)
  compute(0, 0)
  fire_sc(0, 0)
  fire(2, 0)
  wait_gather(1, 1)
  compute(1, 1)
  fire_sc(1, 1)

  def pair(k, c):
    ci = 2 * k
    fire(ci + 1, 1)
    wait_gather(ci, 0)
    wait_sc(ci - 2, 0)
    compute(ci, 0)
    fire_sc(ci, 0)
    fire(ci + 2, 0)
    wait_gather(ci + 1, 1)
    wait_sc(ci - 1, 1)
    compute(ci + 1, 1)
    fire_sc(ci + 1, 1)
    return c

  lax.fori_loop(1, NCHUNK // 2 - 1, pair, 0)
  ci0 = NCHUNK - 2
  fire(ci0 + 1, 1)
  wait_gather(ci0, 0)
  wait_sc(ci0 - 2, 0)
  compute(ci0, 0)
  fire_sc(ci0, 0)
  wait_gather(ci0 + 1, 1)
  wait_sc(ci0 - 1, 1)
  compute(ci0 + 1, 1)
  fire_sc(ci0 + 1, 1)
  wait_sc(ci0, 0)
  wait_sc(ci0 + 1, 1)

  plsc.subcore_barrier()
  pltpu.sync_copy(s_sh.at[pl.ds(sid * RPT, RPT)],
                  s_out.at[cid, pl.ds(sid * RPT, RPT)])


_sc_passA = functools.partial(
    pl.kernel,
    out_type=jax.ShapeDtypeStruct((NC, NP, 16), jnp.float32),
    mesh=_MESH,
    compiler_params=pltpu.CompilerParams(use_tc_tiling_on_sc=False),
    scratch_types=[
        pltpu.VMEM((NCHUNK, B), jnp.int32),
        pltpu.VMEM((NCHUNK, B), jnp.int32),
        pltpu.VMEM((2, B, 16), jnp.float32),
        pltpu.VMEM((2, B, 16), jnp.float32),
        pltpu.VMEM((2, B, 16), jnp.float32),
        pltpu.VMEM((LANES,), jnp.float32),
        pltpu.VMEM_SHARED((NP, 16), jnp.float32),
        pltpu.SemaphoreType.DMA,
        pltpu.SemaphoreType.DMA,
        pltpu.SemaphoreType.DMA,
        pltpu.SemaphoreType.DMA,
    ],
)(_passA_body)


def _make_passB(d_row, n_heads):
  n_chunks = d_row // LANES
  splat_head = [(c if n_heads == HEADS else 0) for c in range(n_chunks)]
  ha_w = d_row + 16

  def body(esrc, edst, HA, avec, T, SP0, SP1, out, src_all, dst_all, ha, ts,
           s0r, s1r, msg, a_v, o_sh, semA, semB, semSA, semSB):
    cid = lax.axis_index("c")
    sid = lax.axis_index("s")
    wid = cid * NS + sid
    has = [ha.at[0], ha.at[1]]
    tss = [ts.at[0], ts.at[1]]
    s0s = [s0r.at[0], s0r.at[1]]
    s1s = [s1r.at[0], s1r.at[1]]
    msgs = [msg.at[0], msg.at[1]]
    sems = [semA, semB]
    ssems = [semSA, semSB]

    def zero_row(i, c):
      for j in range(n_chunks):
        msg[0, i, pl.ds(j * LANES, LANES)] = jnp.zeros((LANES,), jnp.float32)
      return c

    lax.fori_loop(0, B, zero_row, 0)

    def zero_sh(k, c):
      pltpu.sync_copy(msgs[0], o_sh.at[pl.ds(sid * RPT + k * B, B)])
      return c

    lax.fori_loop(0, RPT // B, zero_sh, 0)
    plsc.subcore_barrier()

    pltpu.sync_copy(avec, a_v)
    av = a_v[:]
    pltpu.sync_copy(esrc.at[wid], src_all)
    pltpu.sync_copy(edst.at[wid], dst_all)

    def fire(ci, b):
      pltpu.async_copy(HA.at[src_all.at[ci]], has[b], sems[b])
      pltpu.async_copy(T.at[dst_all.at[ci]], tss[b], sems[b])
      pltpu.async_copy(SP0.at[dst_all.at[ci]], s0s[b], sems[b])
      pltpu.async_copy(SP1.at[dst_all.at[ci]], s1s[b], sems[b])

    def wait_gather(ci, b):
      pltpu.make_async_copy(HA.at[src_all.at[ci]], has[b], sems[b]).wait()
      pltpu.make_async_copy(T.at[dst_all.at[ci]], tss[b], sems[b]).wait()
      pltpu.make_async_copy(SP0.at[dst_all.at[ci]], s0s[b], sems[b]).wait()
      pltpu.make_async_copy(SP1.at[dst_all.at[ci]], s1s[b], sems[b]).wait()

    def compute(ci, b):
      @plsc.parallel_loop(0, B, step=1, unroll=4)
      def per_edge(i):
        tsr = ha[b, i, pl.ds(d_row, LANES)]
        tdr = ts[b, i, :]
        w = _edge_w(tsr, tdr, av)
        srow = s0r[b, i, :] + s1r[b, i, :]
        attn = w / (srow + 1e-16)
        for j in range(n_chunks):
          sp = jnp.take_along_axis(
              attn, jnp.full((LANES,), splat_head[j], jnp.int32), axis=0)
          msg[b, i, pl.ds(j * LANES, LANES)] = (
              ha[b, i, pl.ds(j * LANES, LANES)] * sp)

    def fire_sc(ci, b):
      pltpu.async_copy(msgs[b], o_sh.at[dst_all.at[ci]], ssems[b], add=True)

    def wait_sc(ci, b):
      pltpu.make_async_copy(
          msgs[b], o_sh.at[dst_all.at[ci]], ssems[b]).wait()

    fire(0, 0)
    fire(1, 1)
    wait_gather(0, 0)
    compute(0, 0)
    fire_sc(0, 0)
    fire(2, 0)
    wait_gather(1, 1)
    compute(1, 1)
    fire_sc(1, 1)

    def pair(k, c):
      ci = 2 * k
      fire(ci + 1, 1)
      wait_gather(ci, 0)
      wait_sc(ci - 2, 0)
      compute(ci, 0)
      fire_sc(ci, 0)
      fire(ci + 2, 0)
      wait_gather(ci + 1, 1)
      wait_sc(ci - 1, 1)
      compute(ci + 1, 1)
      fire_sc(ci + 1, 1)
      return c

    lax.fori_loop(1, NCHUNK // 2 - 1, pair, 0)
    ci0 = NCHUNK - 2
    fire(ci0 + 1, 1)
    wait_gather(ci0, 0)
    wait_sc(ci0 - 2, 0)
    compute(ci0, 0)
    fire_sc(ci0, 0)
    wait_gather(ci0 + 1, 1)
    wait_sc(ci0 - 1, 1)
    compute(ci0 + 1, 1)
    fire_sc(ci0 + 1, 1)
    wait_sc(ci0, 0)
    wait_sc(ci0 + 1, 1)

    plsc.subcore_barrier()
    pltpu.sync_copy(o_sh.at[pl.ds(sid * RPT, RPT)],
                    out.at[cid, pl.ds(sid * RPT, RPT)])

  return functools.partial(
      pl.kernel,
      out_type=jax.ShapeDtypeStruct((NC, NP, d_row), jnp.float32),
      mesh=_MESH,
      compiler_params=pltpu.CompilerParams(use_tc_tiling_on_sc=False),
      scratch_types=[
          pltpu.VMEM((NCHUNK, B), jnp.int32),
          pltpu.VMEM((NCHUNK, B), jnp.int32),
          pltpu.VMEM((2, B, ha_w), jnp.float32),
          pltpu.VMEM((2, B, 16), jnp.float32),
          pltpu.VMEM((2, B, 16), jnp.float32),
          pltpu.VMEM((2, B, 16), jnp.float32),
          pltpu.VMEM((2, B, d_row), jnp.float32),
          pltpu.VMEM((LANES,), jnp.float32),
          pltpu.VMEM_SHARED((NP, d_row), jnp.float32),
          pltpu.SemaphoreType.DMA,
          pltpu.SemaphoreType.DMA,
          pltpu.SemaphoreType.DMA,
          pltpu.SemaphoreType.DMA,
      ],
  )(body)


_sc_passB_128 = _make_passB(128, HEADS)


def _build_lr(al, ar, d, ph):
  rows = jnp.arange(d)
  hcol = rows // ph
  lr = jnp.zeros((d, 16), jnp.float32)
  lr = lr.at[rows, hcol].set(al.reshape(-1))
  lr = lr.at[rows, hcol + 8].set(ar.reshape(-1))
  return lr


def _layer(esrc, edst, p, bb_row, W, b_row, LR, d_in, d_out, first, passB):
  HA, T, A = _tc_prep(p, bb_row, W, b_row, LR, d_in, d_out, first)
  Av = A.reshape(16)
  sp = _sc_passA(esrc, edst, T, Av)
  return passB(esrc, edst, HA, Av, T, sp[0], sp[1])


def kernel(x, edge_index, W0, b0, al0, ar0, bb0, W1, b1, al1, ar1, bb1,
           W2, b2, al2, ar2, bb2):
  ei = edge_index.astype(jnp.int32)
  esrc = ei[0].reshape(NW, NCHUNK, B)
  edst = ei[1].reshape(NW, NCHUNK, B)
  x_p = jnp.pad(x, ((0, NP - N), (0, 0)))

  LR0 = _build_lr(al0, ar0, 128, PH)
  LR1 = _build_lr(al1, ar1, 128, PH)
  rows2 = jnp.arange(NCLS)
  LR2 = jnp.zeros((128, 16), jnp.float32)
  for _j in range(8):
    LR2 = LR2.at[rows2, _j].set(al2.reshape(-1))
    LR2 = LR2.at[rows2, 8 + _j].set(ar2.reshape(-1))
  W2p = jnp.pad(W2, ((0, 0), (0, 128 - NCLS)))
  b2p = jnp.pad(b2, (0, 128 - NCLS))
  bb2p = jnp.pad(bb2, (0, 128 - NCLS))

  op0 = _layer(esrc, edst, x_p, b0.reshape(1, -1), W0, b0.reshape(1, -1),
               LR0, 128, 128, True, _sc_passB_128)
  op1 = _layer(esrc, edst, op0, bb0.reshape(1, -1), W1, b1.reshape(1, -1),
               LR1, 128, 128, False, _sc_passB_128)
  op2 = _layer(esrc, edst, op1, bb1.reshape(1, -1), W2p, b2p.reshape(1, -1),
               LR2, 128, 128, False, _sc_passB_128)

  outf = _tc_final(op2, bb2p.reshape(1, -1), 128)
  return outf[:N, :NCLS]


# speedup vs baseline: 1.1011x; 1.1011x over previous
import functools

import jax
import jax.numpy as jnp
from jax import lax
from jax.experimental import pallas as pl
from jax.experimental.pallas import tpu as pltpu
from jax.experimental.pallas import tpu_sc as plsc

N = 10000
E = 320000
D_IN = 128
HEADS = 8
PH = 16
HID = 128
NCLS = 40
SLOPE = 0.2

NC = 2
NS = 16
NW = NC * NS
LANES = 16

NP = 10240
BLK = 256
EW = E // NW
B = 40
NCHUNK = EW // B
BA = 80
NCHA = EW // BA
RPT = NP // NS

_BIG = 1e30


def _leaky(v):
  return jnp.where(v >= 0, v, v * SLOPE)


def _prep_common(h_in, W_ref, b_ref, LR_ref, HA_ref, T_ref, A_ref, d_out):
  h = jnp.dot(h_in, W_ref[...], preferred_element_type=jnp.float32)
  h = h + b_ref[...]
  t = jnp.dot(h, LR_ref[...], preferred_element_type=jnp.float32)
  T_ref[...] = t
  HA_ref[:, 0:d_out] = h
  HA_ref[:, d_out:d_out + 8] = t[:, 0:8]
  HA_ref[:, d_out + 8:d_out + 16] = jnp.zeros((h.shape[0], 8), jnp.float32)
  blockmax = jnp.max(t[:, 0:8], axis=0, keepdims=True)
  cur = jnp.concatenate(
      [blockmax, jnp.full((1, 8), _BIG, jnp.float32)], axis=1)
  i = pl.program_id(0)

  @pl.when(i == 0)
  def _():
    A_ref[...] = cur

  @pl.when(i > 0)
  def _():
    A_ref[...] = jnp.maximum(A_ref[...], cur)


def _tc_prep(p, bb_row, W, b_row, LR, d_in, d_out, first):
  grid = NP // BLK

  def body(p_ref, bb_ref, W_ref, b_ref, LR_ref, HA_ref, T_ref, A_ref):
    if first:
      h_in = p_ref[...]
    else:
      s = p_ref[0] + p_ref[1] + bb_ref[...]
      h_in = jnp.where(s > 0, s, jnp.exp(s) - 1.0)
    _prep_common(h_in, W_ref, b_ref, LR_ref, HA_ref, T_ref, A_ref, d_out)

  in_spec_p = (
      pl.BlockSpec((BLK, d_in), lambda i: (i, 0)) if first
      else pl.BlockSpec((2, BLK, d_in), lambda i: (0, i, 0)))
  return pl.pallas_call(
      body,
      grid=(grid,),
      in_specs=[
          in_spec_p,
          pl.BlockSpec((1, d_in), lambda i: (0, 0)),
          pl.BlockSpec((d_in, d_out), lambda i: (0, 0)),
          pl.BlockSpec((1, d_out), lambda i: (0, 0)),
          pl.BlockSpec((d_out, 16), lambda i: (0, 0)),
      ],
      out_specs=[
          pl.BlockSpec((BLK, d_out + 16), lambda i: (i, 0)),
          pl.BlockSpec((BLK, 16), lambda i: (i, 0)),
          pl.BlockSpec((1, 16), lambda i: (0, 0)),
      ],
      out_shape=[
          jax.ShapeDtypeStruct((NP, d_out + 16), jnp.float32),
          jax.ShapeDtypeStruct((NP, 16), jnp.float32),
          jax.ShapeDtypeStruct((1, 16), jnp.float32),
      ],
  )(p, bb_row, W, b_row, LR)


def _final_body(p_ref, bb_ref, o_ref):
  o_ref[...] = p_ref[0] + p_ref[1] + bb_ref[...]


def _tc_final(p, bb_row, d_out):
  grid = NP // BLK
  return pl.pallas_call(
      _final_body,
      grid=(grid,),
      in_specs=[
          pl.BlockSpec((2, BLK, d_out), lambda i: (0, i, 0)),
          pl.BlockSpec((1, d_out), lambda i: (0, 0)),
      ],
      out_specs=pl.BlockSpec((BLK, d_out), lambda i: (i, 0)),
      out_shape=jax.ShapeDtypeStruct((NP, d_out), jnp.float32),
  )(p, bb_row)


_MESH = plsc.VectorSubcoreMesh(core_axis_name="c", subcore_axis_name="s")


def _edge_w(ts, td, av):
  rot_idx = (lax.iota(jnp.int32, LANES) & 7) + 8
  rot = jnp.take_along_axis(td, rot_idx, axis=0)
  e = _leaky(ts + rot)
  m = _leaky(rot + av)
  return jnp.exp(e - m)


def _passA_body(esrc, edst, T, avec, s_out, src_all, dst_all, tsrc, tdst,
                wbuf, a_v, s_sh, semA, semB, semSA, semSB):
  cid = lax.axis_index("c")
  sid = lax.axis_index("s")
  wid = cid * NS + sid
  tsrcs = [tsrc.at[0], tsrc.at[1]]
  tdsts = [tdst.at[0], tdst.at[1]]
  wbufs = [wbuf.at[0], wbuf.at[1]]
  sems = [semA, semB]
  ssems = [semSA, semSB]

  def zero_row(i, c):
    wbuf[0, i, :] = jnp.zeros((LANES,), jnp.float32)
    return c

  lax.fori_loop(0, BA, zero_row, 0)

  def zero_sh(k, c):
    pltpu.sync_copy(wbufs[0], s_sh.at[pl.ds(sid * RPT + k * BA, BA)])
    return c

  lax.fori_loop(0, RPT // BA, zero_sh, 0)
  plsc.subcore_barrier()

  pltpu.sync_copy(avec, a_v)
  av = a_v[:]
  pltpu.sync_copy(esrc.at[wid], src_all)
  pltpu.sync_copy(edst.at[wid], dst_all)

  def fire(ci, b):
    pltpu.async_copy(T.at[src_all.at[ci]], tsrcs[b], sems[b])
    pltpu.async_copy(T.at[dst_all.at[ci]], tdsts[b], sems[b])

  def wait_gather(ci, b):
    pltpu.make_async_copy(T.at[src_all.at[ci]], tsrcs[b], sems[b]).wait()
    pltpu.make_async_copy(T.at[dst_all.at[ci]], tdsts[b], sems[b]).wait()

  def compute(ci, b):
    @plsc.parallel_loop(0, BA, step=1, unroll=4)
    def per_edge(i):
      wbuf[b, i, :] = _edge_w(tsrc[b, i, :], tdst[b, i, :], av)

  def fire_sc(ci, b):
    pltpu.async_copy(wbufs[b], s_sh.at[dst_all.at[ci]], ssems[b], add=True)

  def wait_sc(ci, b):
    pltpu.make_async_copy(wbufs[b], s_sh.at[dst_all.at[ci]], ssems[b]).wait()

  fire(0, 0)
  fire(1, 1)
  wait_gather(0, 0)
  compute(0, 0)
  fire_sc(0, 0)
  fire(2, 0)
  wait_gather(1, 1)
  compute(1, 1)
  fire_sc(1, 1)

  def pair(k, c):
    ci = 2 * k
    fire(ci + 1, 1)
    wait_gather(ci, 0)
    wait_sc(ci - 2, 0)
    compute(ci, 0)
    fire_sc(ci, 0)
    fire(ci + 2, 0)
    wait_gather(ci + 1, 1)
    wait_sc(ci - 1, 1)
    compute(ci + 1, 1)
    fire_sc(ci + 1, 1)
    return c

  lax.fori_loop(1, (NCHA - 1) // 2, pair, 0)
  ci0 = NCHA - 1
  wait_gather(ci0, 0)
  wait_sc(ci0 - 2, 0)
  compute(ci0, 0)
  fire_sc(ci0, 0)
  wait_sc(ci0 - 1, 1)
  wait_sc(ci0, 0)

  plsc.subcore_barrier()
  pltpu.sync_copy(s_sh.at[pl.ds(sid * RPT, RPT)],
                  s_out.at[cid, pl.ds(sid * RPT, RPT)])


_sc_passA = functools.partial(
    pl.kernel,
    out_type=jax.ShapeDtypeStruct((NC, NP, 16), jnp.float32),
    mesh=_MESH,
    compiler_params=pltpu.CompilerParams(use_tc_tiling_on_sc=False),
    scratch_types=[
        pltpu.VMEM((NCHA, BA), jnp.int32),
        pltpu.VMEM((NCHA, BA), jnp.int32),
        pltpu.VMEM((2, BA, 16), jnp.float32),
        pltpu.VMEM((2, BA, 16), jnp.float32),
        pltpu.VMEM((2, BA, 16), jnp.float32),
        pltpu.VMEM((LANES,), jnp.float32),
        pltpu.VMEM_SHARED((NP, 16), jnp.float32),
        pltpu.SemaphoreType.DMA,
        pltpu.SemaphoreType.DMA,
        pltpu.SemaphoreType.DMA,
        pltpu.SemaphoreType.DMA,
    ],
)(_passA_body)


def _make_passB(d_row, n_heads):
  n_chunks = d_row // LANES
  splat_head = [(c if n_heads == HEADS else 0) for c in range(n_chunks)]
  ha_w = d_row + 16

  def body(esrc, edst, HA, avec, T, SP0, SP1, out, src_all, dst_all, ha, ts,
           s0r, s1r, msg, a_v, o_sh, semA, semB, semSA, semSB):
    cid = lax.axis_index("c")
    sid = lax.axis_index("s")
    wid = cid * NS + sid
    has = [ha.at[0], ha.at[1]]
    tss = [ts.at[0], ts.at[1]]
    s0s = [s0r.at[0], s0r.at[1]]
    s1s = [s1r.at[0], s1r.at[1]]
    msgs = [msg.at[0], msg.at[1]]
    sems = [semA, semB]
    ssems = [semSA, semSB]

    def zero_row(i, c):
      for j in range(n_chunks):
        msg[0, i, pl.ds(j * LANES, LANES)] = jnp.zeros((LANES,), jnp.float32)
      return c

    lax.fori_loop(0, B, zero_row, 0)

    def zero_sh(k, c):
      pltpu.sync_copy(msgs[0], o_sh.at[pl.ds(sid * RPT + k * B, B)])
      return c

    lax.fori_loop(0, RPT // B, zero_sh, 0)
    plsc.subcore_barrier()

    pltpu.sync_copy(avec, a_v)
    av = a_v[:]
    pltpu.sync_copy(esrc.at[wid], src_all)
    pltpu.sync_copy(edst.at[wid], dst_all)

    def fire(ci, b):
      pltpu.async_copy(HA.at[src_all.at[ci]], has[b], sems[b])
      pltpu.async_copy(T.at[dst_all.at[ci]], tss[b], sems[b])
      pltpu.async_copy(SP0.at[dst_all.at[ci]], s0s[b], sems[b])
      pltpu.async_copy(SP1.at[dst_all.at[ci]], s1s[b], sems[b])

    def wait_gather(ci, b):
      pltpu.make_async_copy(HA.at[src_all.at[ci]], has[b], sems[b]).wait()
      pltpu.make_async_copy(T.at[dst_all.at[ci]], tss[b], sems[b]).wait()
      pltpu.make_async_copy(SP0.at[dst_all.at[ci]], s0s[b], sems[b]).wait()
      pltpu.make_async_copy(SP1.at[dst_all.at[ci]], s1s[b], sems[b]).wait()

    def compute(ci, b):
      @plsc.parallel_loop(0, B, step=1, unroll=4)
      def per_edge(i):
        tsr = ha[b, i, pl.ds(d_row, LANES)]
        tdr = ts[b, i, :]
        w = _edge_w(tsr, tdr, av)
        srow = s0r[b, i, :] + s1r[b, i, :]
        attn = w / (srow + 1e-16)
        for j in range(n_chunks):
          sp = jnp.take_along_axis(
              attn, jnp.full((LANES,), splat_head[j], jnp.int32), axis=0)
          msg[b, i, pl.ds(j * LANES, LANES)] = (
              ha[b, i, pl.ds(j * LANES, LANES)] * sp)

    def fire_sc(ci, b):
      pltpu.async_copy(msgs[b], o_sh.at[dst_all.at[ci]], ssems[b], add=True)

    def wait_sc(ci, b):
      pltpu.make_async_copy(
          msgs[b], o_sh.at[dst_all.at[ci]], ssems[b]).wait()

    fire(0, 0)
    fire(1, 1)
    wait_gather(0, 0)
    compute(0, 0)
    fire_sc(0, 0)
    fire(2, 0)
    wait_gather(1, 1)
    compute(1, 1)
    fire_sc(1, 1)

    def pair(k, c):
      ci = 2 * k
      fire(ci + 1, 1)
      wait_gather(ci, 0)
      wait_sc(ci - 2, 0)
      compute(ci, 0)
      fire_sc(ci, 0)
      fire(ci + 2, 0)
      wait_gather(ci + 1, 1)
      wait_sc(ci - 1, 1)
      compute(ci + 1, 1)
      fire_sc(ci + 1, 1)
      return c

    lax.fori_loop(1, NCHUNK // 2 - 1, pair, 0)
    ci0 = NCHUNK - 2
    fire(ci0 + 1, 1)
    wait_gather(ci0, 0)
    wait_sc(ci0 - 2, 0)
    compute(ci0, 0)
    fire_sc(ci0, 0)
    wait_gather(ci0 + 1, 1)
    wait_sc(ci0 - 1, 1)
    compute(ci0 + 1, 1)
    fire_sc(ci0 + 1, 1)
    wait_sc(ci0, 0)
    wait_sc(ci0 + 1, 1)

    plsc.subcore_barrier()
    pltpu.sync_copy(o_sh.at[pl.ds(sid * RPT, RPT)],
                    out.at[cid, pl.ds(sid * RPT, RPT)])

  return functools.partial(
      pl.kernel,
      out_type=jax.ShapeDtypeStruct((NC, NP, d_row), jnp.float32),
      mesh=_MESH,
      compiler_params=pltpu.CompilerParams(use_tc_tiling_on_sc=False),
      scratch_types=[
          pltpu.VMEM((NCHUNK, B), jnp.int32),
          pltpu.VMEM((NCHUNK, B), jnp.int32),
          pltpu.VMEM((2, B, ha_w), jnp.float32),
          pltpu.VMEM((2, B, 16), jnp.float32),
          pltpu.VMEM((2, B, 16), jnp.float32),
          pltpu.VMEM((2, B, 16), jnp.float32),
          pltpu.VMEM((2, B, d_row), jnp.float32),
          pltpu.VMEM((LANES,), jnp.float32),
          pltpu.VMEM_SHARED((NP, d_row), jnp.float32),
          pltpu.SemaphoreType.DMA,
          pltpu.SemaphoreType.DMA,
          pltpu.SemaphoreType.DMA,
          pltpu.SemaphoreType.DMA,
      ],
  )(body)


_sc_passB_128 = _make_passB(128, HEADS)


def _build_lr(al, ar, d, ph):
  rows = jnp.arange(d)
  hcol = rows // ph
  lr = jnp.zeros((d, 16), jnp.float32)
  lr = lr.at[rows, hcol].set(al.reshape(-1))
  lr = lr.at[rows, hcol + 8].set(ar.reshape(-1))
  return lr


def _layer(eidx, p, bb_row, W, b_row, LR, d_in, d_out, first, passB):
  esrc, edst, esrcA, edstA = eidx
  HA, T, A = _tc_prep(p, bb_row, W, b_row, LR, d_in, d_out, first)
  Av = A.reshape(16)
  sp = _sc_passA(esrcA, edstA, T, Av)
  return passB(esrc, edst, HA, Av, T, sp[0], sp[1])


def kernel(x, edge_index, W0, b0, al0, ar0, bb0, W1, b1, al1, ar1, bb1,
           W2, b2, al2, ar2, bb2):
  ei = edge_index.astype(jnp.int32)
  esrc = ei[0].reshape(NW, NCHUNK, B)
  edst = ei[1].reshape(NW, NCHUNK, B)
  esrcA = ei[0].reshape(NW, NCHA, BA)
  edstA = ei[1].reshape(NW, NCHA, BA)
  x_p = jnp.pad(x, ((0, NP - N), (0, 0)))

  LR0 = _build_lr(al0, ar0, 128, PH)
  LR1 = _build_lr(al1, ar1, 128, PH)
  rows2 = jnp.arange(NCLS)
  LR2 = jnp.zeros((128, 16), jnp.float32)
  for _j in range(8):
    LR2 = LR2.at[rows2, _j].set(al2.reshape(-1))
    LR2 = LR2.at[rows2, 8 + _j].set(ar2.reshape(-1))
  W2p = jnp.pad(W2, ((0, 0), (0, 128 - NCLS)))
  b2p = jnp.pad(b2, (0, 128 - NCLS))
  bb2p = jnp.pad(bb2, (0, 128 - NCLS))

  eidx = (esrc, edst, esrcA, edstA)
  op0 = _layer(eidx, x_p, b0.reshape(1, -1), W0, b0.reshape(1, -1),
               LR0, 128, 128, True, _sc_passB_128)
  op1 = _layer(eidx, op0, bb0.reshape(1, -1), W1, b1.reshape(1, -1),
               LR1, 128, 128, False, _sc_passB_128)
  op2 = _layer(eidx, op1, bb1.reshape(1, -1), W2p, b2p.reshape(1, -1),
               LR2, 128, 128, False, _sc_passB_128)

  outf = _tc_final(op2, bb2p.reshape(1, -1), 128)
  return outf[:N, :NCLS]
